# Initial kernel scaffold; baseline (speedup 1.0000x reference)
import jax
import jax.numpy as jnp
from jax.experimental import pallas as pl


def kernel(x, emb):
    raise NotImplementedError("write your pallas kernel here")


# speedup vs baseline: 1.9711x; 1.9711x over previous
import jax
import jax.numpy as jnp
from jax.experimental import pallas as pl

_CH_BLK = 256
_LEN_BLK = 512


def _add_pos_kernel(x_ref, emb_ref, out_ref):
    pos = jnp.transpose(emb_ref[...], (1, 0))
    out_ref[...] = x_ref[...] + pos[None, :, :]


def kernel(x, emb):
    batch, ch, length = x.shape
    grid = (ch // _CH_BLK, length // _LEN_BLK)
    return pl.pallas_call(
        _add_pos_kernel,
        grid=grid,
        in_specs=[
            pl.BlockSpec((batch, _CH_BLK, _LEN_BLK), lambda i, j: (0, i, j)),
            pl.BlockSpec((_LEN_BLK, _CH_BLK), lambda i, j: (j, i)),
        ],
        out_specs=pl.BlockSpec((batch, _CH_BLK, _LEN_BLK), lambda i, j: (0, i, j)),
        out_shape=jax.ShapeDtypeStruct(x.shape, x.dtype),
    )(x, emb)


# speedup vs baseline: 2.1634x; 1.0975x over previous
import jax
import jax.numpy as jnp
from jax.experimental import pallas as pl

_CH_BLK = 512
_LEN_BLK = 1024


def _add_pos_kernel(x_ref, emb_ref, out_ref):
    pos = jnp.transpose(emb_ref[...], (1, 0))
    out_ref[...] = x_ref[...] + pos[None, :, :]


def kernel(x, emb):
    batch, ch, length = x.shape
    grid = (ch // _CH_BLK, length // _LEN_BLK)
    return pl.pallas_call(
        _add_pos_kernel,
        grid=grid,
        in_specs=[
            pl.BlockSpec((batch, _CH_BLK, _LEN_BLK), lambda i, j: (0, i, j)),
            pl.BlockSpec((_LEN_BLK, _CH_BLK), lambda i, j: (j, i)),
        ],
        out_specs=pl.BlockSpec((batch, _CH_BLK, _LEN_BLK), lambda i, j: (0, i, j)),
        out_shape=jax.ShapeDtypeStruct(x.shape, x.dtype),
    )(x, emb)


# speedup vs baseline: 2.2205x; 1.0264x over previous
import jax
import jax.numpy as jnp
from jax.experimental import pallas as pl

_CH_BLK = 256
_LEN_BLK = 2048


def _add_pos_kernel(x_ref, emb_ref, out_ref):
    pos = jnp.transpose(emb_ref[...], (1, 0))
    out_ref[...] = x_ref[...] + pos[None, :, :]


def kernel(x, emb):
    batch, ch, length = x.shape
    grid = (ch // _CH_BLK, length // _LEN_BLK)
    return pl.pallas_call(
        _add_pos_kernel,
        grid=grid,
        in_specs=[
            pl.BlockSpec((batch, _CH_BLK, _LEN_BLK), lambda i, j: (0, i, j)),
            pl.BlockSpec((_LEN_BLK, _CH_BLK), lambda i, j: (j, i)),
        ],
        out_specs=pl.BlockSpec((batch, _CH_BLK, _LEN_BLK), lambda i, j: (0, i, j)),
        out_shape=jax.ShapeDtypeStruct(x.shape, x.dtype),
    )(x, emb)


# speedup vs baseline: 2.2569x; 1.0164x over previous
import jax
import jax.numpy as jnp
from jax.experimental import pallas as pl

_CH_BLK = 128
_LEN_BLK = 4096


def _add_pos_kernel(x_ref, emb_ref, out_ref):
    pos = jnp.transpose(emb_ref[...], (1, 0))
    out_ref[...] = x_ref[...] + pos[None, :, :]


def kernel(x, emb):
    batch, ch, length = x.shape
    grid = (ch // _CH_BLK, length // _LEN_BLK)
    return pl.pallas_call(
        _add_pos_kernel,
        grid=grid,
        in_specs=[
            pl.BlockSpec((batch, _CH_BLK, _LEN_BLK), lambda i, j: (0, i, j)),
            pl.BlockSpec((_LEN_BLK, _CH_BLK), lambda i, j: (j, i)),
        ],
        out_specs=pl.BlockSpec((batch, _CH_BLK, _LEN_BLK), lambda i, j: (0, i, j)),
        out_shape=jax.ShapeDtypeStruct(x.shape, x.dtype),
    )(x, emb)
